# baseline (speedup 1.0000x reference)
import jax
import jax.numpy as jnp
from jax import lax
from jax.experimental import pallas as pl
from jax.experimental.pallas import tpu as pltpu
from jax.experimental.pallas import tpu_sc as plsc

NC, NS, L = 2, 16, 16
B, T, D = 8, 512, 512
MAX_LEN = 4096
QW = 4
G = 32
NTB = MAX_LEN // G
GPL = G // L
MAXG = (NTB + QW - 1) // QW
ZR = 16
NB = 3
S = 24


def _expand_body(xflat, dur, out, mel,
                 dur_v, idxarr, idxf, idxq, xbuf, obuf, zbuf, melv,
                 gsem, wsem, zsem):
    wid = lax.axis_index("s") * NC + lax.axis_index("c")
    b = wid // QW
    q = (wid + b) % QW
    obase = b * MAX_LEN

    pltpu.sync_copy(dur.at[b], dur_v)

    zv = jnp.zeros((L,), jnp.int32)
    zvf = jnp.zeros((L,), jnp.float32)

    def init_body(k, c):
        idxarr[pl.ds(k * L, L)] = zv
        return c
    lax.fori_loop(0, MAX_LEN // L, init_body, 0)

    def zb_body(rr, c):
        for kk in range(D // L):
            zbuf[rr, pl.ds(kk * L, L)] = zvf
        return c
    lax.fori_loop(0, ZR, zb_body, 0)

    def p1(k, csum_base):
        v = dur_v[pl.ds(k * L, L)]
        c = plsc.cumsum(v) + csum_base
        start = c - v
        t = lax.iota(jnp.int32, L) + k * L
        m = (v > 0) & (start < MAX_LEN)
        plsc.store_scatter(idxarr, [start], t, mask=m)
        return jnp.max(c)
    total = lax.fori_loop(0, T // L, p1, jnp.int32(0))

    nvt = (total + (G - 1)) // G
    nzt = NTB - nvt
    gs = q * nvt // QW
    ge = (q + 1) * nvt // QW
    ng = ge - gs
    zs = nvt + q * nzt // QW
    ze = nvt + (q + 1) * nzt // QW

    def zfire(j, c):
        for h in range(G // ZR):
            pltpu.async_copy(zbuf,
                             out.at[pl.ds(obase + j * G + h * ZR, ZR)], zsem)
        return c
    lax.fori_loop(zs, ze, zfire, 0)

    def p2a(k, mv):
        return jnp.maximum(mv, idxarr[pl.ds(k * L, L)])
    m0 = jnp.max(lax.fori_loop(0, gs * GPL, p2a, zv))

    def p2b(k, m):
        kk = gs * GPL + k
        v = idxarr[pl.ds(kk * L, L)]
        c = jnp.maximum(plsc.cummax(v), m)
        idxf[pl.ds(k * L, L)] = c
        return jnp.max(c)
    lax.fori_loop(0, ng * GPL, p2b, m0)

    def tile_info(jl):
        v0 = idxf[pl.ds(jl * G, L)]
        v1 = idxf[pl.ds(jl * G + L, L)]
        fs = pl.multiple_of(
            jnp.minimum(jnp.min(v0) & -8, T - S), 8)
        fits = (jnp.max(v1) - fs) < S
        return fs, fits

    def gstart(jl):
        s = jl % NB
        fs, fits = tile_info(jl)

        @pl.when(fits)
        def _():
            pltpu.async_copy(xflat.at[pl.ds(b * T + fs, S)],
                             xbuf.at[pl.ds(s * S, S)], gsem.at[s])

        @pl.when(jnp.logical_not(fits))
        def _():
            for k2 in range(GPL):
                idxq[pl.ds(s * G + k2 * L, L)] = (
                    idxf[pl.ds(jl * G + k2 * L, L)] + b * T)
            pltpu.async_copy(xflat.at[idxq.at[pl.ds(s * G, G)]],
                             obuf.at[pl.ds(s * G, G)], gsem.at[s])

    def prime(i, c):
        gstart(i)
        return c
    lax.fori_loop(0, jnp.minimum(NB, ng), prime, 0)

    def body(jl, c):
        cur = jl % NB

        @pl.when((jl >= 1) & (jl + NB - 1 < ng))
        def _():
            pltpu.make_async_copy(
                obuf.at[pl.ds(0, G)], out.at[pl.ds(obase, G)],
                wsem.at[(jl - 1) % NB]).wait()
            gstart(jl + NB - 1)

        fs, fits = tile_info(jl)

        @pl.when(fits)
        def _():
            pltpu.make_async_copy(
                xflat.at[pl.ds(0, S)], xbuf.at[pl.ds(0, S)],
                gsem.at[cur]).wait()
            for k2 in range(GPL):
                srv = (idxf[pl.ds(jl * G + k2 * L, L)] - fs) + cur * S
                rowv = lax.iota(jnp.int32, L) + (cur * G + k2 * L)
                col = jnp.zeros((L,), jnp.int32)
                one = jnp.ones((L,), jnp.int32)
                for _c in range(D):
                    val = plsc.load_gather(xbuf, [srv, col])
                    plsc.store_scatter(obuf, [rowv, col], val)
                    col = col + one

        @pl.when(jnp.logical_not(fits))
        def _():
            pltpu.make_async_copy(
                xflat.at[pl.ds(0, G)], obuf.at[pl.ds(0, G)],
                gsem.at[cur]).wait()

        vt = jnp.clip(total - (gs + jl) * G, 0, G)

        @pl.when(vt < G)
        def _():
            def zr(rr, c2):
                for kk in range(D // L):
                    obuf[cur * G + rr, pl.ds(kk * L, L)] = zvf
                return c2
            lax.fori_loop(vt, G, zr, 0)

        pltpu.async_copy(obuf.at[pl.ds(cur * G, G)],
                         out.at[pl.ds(obase + (gs + jl) * G, G)],
                         wsem.at[cur])
        return c
    lax.fori_loop(0, ng, body, 0)

    def wdrain(i, c):
        pltpu.make_async_copy(
            obuf.at[pl.ds(0, G)], out.at[pl.ds(obase, G)],
            wsem.at[i % NB]).wait()
        return c
    lax.fori_loop(jnp.maximum(ng - NB, 0), ng, wdrain, 0)

    def zdrain(j, c):
        pltpu.make_async_copy(zbuf, out.at[pl.ds(obase, ZR)], zsem).wait()
        return c
    lax.fori_loop(0, (ze - zs) * (G // ZR), zdrain, 0)

    @pl.when(q == 0)
    def _():
        melv[...] = jnp.broadcast_to(total, (L,))
        pltpu.sync_copy(melv, mel.at[b])


_sc_expand = pl.kernel(
    _expand_body,
    out_type=(jax.ShapeDtypeStruct((B * MAX_LEN, D), jnp.float32),
              jax.ShapeDtypeStruct((B, L), jnp.int32)),
    mesh=plsc.VectorSubcoreMesh(core_axis_name="c", subcore_axis_name="s",
                                num_cores=NC, num_subcores=NS),
    compiler_params=pltpu.CompilerParams(needs_layout_passes=False),
    scratch_types=[
        pltpu.VMEM((T,), jnp.int32),
        pltpu.VMEM((MAX_LEN,), jnp.int32),
        pltpu.VMEM((MAXG * G,), jnp.int32),
        pltpu.VMEM((NB * G,), jnp.int32),
        pltpu.VMEM((NB * S, D), jnp.float32),
        pltpu.VMEM((NB * G, D), jnp.float32),
        pltpu.VMEM((ZR, D), jnp.float32),
        pltpu.VMEM((L,), jnp.int32),
        pltpu.SemaphoreType.DMA((NB,)),
        pltpu.SemaphoreType.DMA((NB,)),
        pltpu.SemaphoreType.DMA,
    ],
)


def kernel(x, duration, max_len):
    outflat, mel16 = _sc_expand(x.reshape(B * T, D), duration)
    return outflat.reshape(B, MAX_LEN, D), mel16[:, 0]


# speedup vs baseline: 4.2076x; 4.2076x over previous
import jax
import jax.numpy as jnp
from jax import lax
from jax.experimental import pallas as pl
from jax.experimental.pallas import tpu as pltpu
from jax.experimental.pallas import tpu_sc as plsc

NC, NS, L = 2, 16, 16
NW = NC * NS
B, T, D = 8, 512, 512
MAX_LEN = 4096
QW = 4
G = 64
NTB = MAX_LEN // G
GPL = G // L
MAXG = (NTB + QW - 1) // QW
ZR = 16
NB = 3


def _expand_body(xflat, dur, out, mel,
                 dur_v, idxarr, idxg, gbuf, zbuf, melv, gsem, wsem, zsem):
    wid = lax.axis_index("s") * NC + lax.axis_index("c")
    b = wid // QW
    q = (wid + b) % QW
    obase = b * MAX_LEN

    pltpu.sync_copy(dur.at[b], dur_v)

    zv = jnp.zeros((L,), jnp.int32)
    zvf = jnp.zeros((L,), jnp.float32)

    def init_body(k, c):
        idxarr[pl.ds(k * L, L)] = zv
        return c
    lax.fori_loop(0, MAX_LEN // L, init_body, 0)

    def zb_body(r, c):
        for kk in range(D // L):
            zbuf[r, pl.ds(kk * L, L)] = zvf
        return c
    lax.fori_loop(0, ZR, zb_body, 0)

    def p1(k, csum_base):
        v = dur_v[pl.ds(k * L, L)]
        c = plsc.cumsum(v) + csum_base
        start = c - v
        t = lax.iota(jnp.int32, L) + k * L
        m = (v > 0) & (start < MAX_LEN)
        plsc.store_scatter(idxarr, [start], t, mask=m)
        return jnp.max(c)
    total = lax.fori_loop(0, T // L, p1, jnp.int32(0))

    nvt = (total + (G - 1)) // G
    nzt = NTB - nvt
    gs = q * nvt // QW
    ge = (q + 1) * nvt // QW
    ng = ge - gs
    zs = nvt + q * nzt // QW
    ze = nvt + (q + 1) * nzt // QW

    def zfire(j, c):
        for h in range(G // ZR):
            pltpu.async_copy(zbuf,
                             out.at[pl.ds(obase + j * G + h * ZR, ZR)], zsem)
        return c
    lax.fori_loop(zs, ze, zfire, 0)

    def p2a(k, mv):
        return jnp.maximum(mv, idxarr[pl.ds(k * L, L)])
    m0 = jnp.max(lax.fori_loop(0, gs * GPL, p2a, zv))

    def p2b(k, m):
        kk = gs * GPL + k
        v = idxarr[pl.ds(kk * L, L)]
        c = jnp.maximum(plsc.cummax(v), m)
        p = kk * L + lax.iota(jnp.int32, L)
        idxg[pl.ds(k * L, L)] = jnp.where(p < total, b * T + c, 0)
        return jnp.max(c)
    lax.fori_loop(0, ng * GPL, p2b, m0)

    def gstart(jl):
        s = jl % NB
        pltpu.async_copy(xflat.at[idxg.at[pl.ds(jl * G, G)]],
                         gbuf.at[s], gsem.at[s])

    def prime(i, c):
        gstart(i)
        return c
    lax.fori_loop(0, jnp.minimum(NB, ng), prime, 0)

    def body(jl, c):
        cur = jl % NB

        @pl.when((jl >= 1) & (jl + NB - 1 < ng))
        def _():
            pltpu.make_async_copy(
                gbuf.at[(jl - 1) % NB], out.at[pl.ds(obase, G)],
                wsem.at[(jl - 1) % NB]).wait()
            gstart(jl + NB - 1)

        pltpu.make_async_copy(
            xflat.at[pl.ds(0, G)], gbuf.at[cur], gsem.at[cur]).wait()

        vt = jnp.clip(total - (gs + jl) * G, 0, G)

        @pl.when(vt < G)
        def _():
            def zr(r, c2):
                for kk in range(D // L):
                    gbuf[cur, r, pl.ds(kk * L, L)] = zvf
                return c2
            lax.fori_loop(vt, G, zr, 0)

        pltpu.async_copy(gbuf.at[cur], out.at[pl.ds(obase + (gs + jl) * G, G)],
                         wsem.at[cur])
        return c
    lax.fori_loop(0, ng, body, 0)

    def wdrain(i, c):
        pltpu.make_async_copy(
            gbuf.at[i % NB], out.at[pl.ds(obase, G)], wsem.at[i % NB]).wait()
        return c
    lax.fori_loop(jnp.maximum(ng - NB, 0), ng, wdrain, 0)

    def zdrain(j, c):
        pltpu.make_async_copy(zbuf, out.at[pl.ds(obase, ZR)], zsem).wait()
        return c
    lax.fori_loop(0, (ze - zs) * (G // ZR), zdrain, 0)

    @pl.when(q == 0)
    def _():
        melv[...] = jnp.broadcast_to(total, (L,))
        pltpu.sync_copy(melv, mel.at[b])


_sc_expand = pl.kernel(
    _expand_body,
    out_type=(jax.ShapeDtypeStruct((B * MAX_LEN, D), jnp.float32),
              jax.ShapeDtypeStruct((B, L), jnp.int32)),
    mesh=plsc.VectorSubcoreMesh(core_axis_name="c", subcore_axis_name="s",
                                num_cores=NC, num_subcores=NS),
    compiler_params=pltpu.CompilerParams(needs_layout_passes=False),
    scratch_types=[
        pltpu.VMEM((T,), jnp.int32),
        pltpu.VMEM((MAX_LEN,), jnp.int32),
        pltpu.VMEM((MAXG * G,), jnp.int32),
        pltpu.VMEM((NB, G, D), jnp.float32),
        pltpu.VMEM((ZR, D), jnp.float32),
        pltpu.VMEM((L,), jnp.int32),
        pltpu.SemaphoreType.DMA((NB,)),
        pltpu.SemaphoreType.DMA((NB,)),
        pltpu.SemaphoreType.DMA,
    ],
)


def kernel(x, duration, max_len):
    outflat, mel16 = _sc_expand(x.reshape(B * T, D), duration)
    return outflat.reshape(B, MAX_LEN, D), mel16[:, 0]
